# baseline (speedup 1.0000x reference)
import functools

import jax
import jax.numpy as jnp
from jax.experimental import pallas as pl
from jax.experimental.pallas import tpu as pltpu

_EPS = 1e-5


def _gn_relu(a, gamma, beta, groups):
    c, tt = a.shape
    gs = c // groups
    rs = jnp.sum(a, axis=1, keepdims=True)
    rq = jnp.sum(a * a, axis=1, keepdims=True)
    gi = jax.lax.broadcasted_iota(jnp.int32, (groups, c), 0)
    gc = jax.lax.broadcasted_iota(jnp.int32, (groups, c), 1) // gs
    gind = (gi == gc).astype(jnp.float32)
    ci = jax.lax.broadcasted_iota(jnp.int32, (c, groups), 0) // gs
    cg = jax.lax.broadcasted_iota(jnp.int32, (c, groups), 1)
    gind_t = (ci == cg).astype(jnp.float32)
    cnt = jnp.float32(gs * tt)
    gmean = jnp.dot(gind, rs, preferred_element_type=jnp.float32) / cnt
    gsq = jnp.dot(gind, rq, preferred_element_type=jnp.float32) / cnt
    ginv = jax.lax.rsqrt(gsq - gmean * gmean + _EPS)
    mean_c = jnp.dot(gind_t, gmean, preferred_element_type=jnp.float32)
    inv_c = jnp.dot(gind_t, ginv, preferred_element_type=jnp.float32)
    sc = gamma * inv_c
    of = beta - mean_c * sc
    return jnp.maximum(a * sc + of, 0.0)


def _fused_kernel(flf_ref, x_ref, wcur_ref, wlr_ref, wprop_ref,
                  bcur_ref, gcur_ref, becur_ref, blr_ref, glr_ref, belr_ref,
                  bprop_ref, gprop_ref, beprop_ref,
                  mixed_ref, feat_ref, *, t, pc, pc2):
    flf = flf_ref[...]
    T = flf.shape[1]
    K = 128
    R = T // K
    m1 = jnp.max(flf, axis=0, keepdims=True)
    m = jnp.concatenate(
        [m1[:, j * K:(j + 1) * K] for j in range(R)], axis=0)
    mn = m / jnp.sum(m)
    ku = jax.lax.broadcasted_iota(jnp.int32, (K, K), 0)
    kv = jax.lax.broadcasted_iota(jnp.int32, (K, K), 1)
    upper = (ku <= kv).astype(jnp.float32)
    rowcum = jnp.dot(mn, upper, preferred_element_type=jnp.float32)
    ru = jax.lax.broadcasted_iota(jnp.int32, (R, R), 0)
    rv = jax.lax.broadcasted_iota(jnp.int32, (R, R), 1)
    strict_lower = (rv < ru).astype(jnp.float32)
    rowtot = jnp.sum(mn, axis=1, keepdims=True)
    prev = jnp.dot(strict_lower, rowtot, preferred_element_type=jnp.float32)
    cdf_i = ((rowcum + prev) * jnp.float32(t)).astype(jnp.int32)
    sentinel = jnp.int32(jnp.iinfo(jnp.int32).max)
    cur = jnp.min(jnp.where(cdf_i >= 0, cdf_i, sentinel))
    lin = (jax.lax.broadcasted_iota(jnp.int32, (R, K), 0) * K
           + jax.lax.broadcasted_iota(jnp.int32, (R, K), 1))
    big = jnp.int32(1 << 30)
    hit = jnp.min(jnp.where(cdf_i == cur, lin, big))
    first_idx = jnp.where(hit == big, jnp.int32(0), hit)
    lin2 = jax.lax.broadcasted_iota(jnp.int32, (T, 1), 0)
    onehot = (lin2 == first_idx).astype(jnp.float32)
    col = jnp.dot(flf, onehot, preferred_element_type=jnp.float32)

    x = x_ref[...]
    a1 = jnp.dot(wcur_ref[...], x,
                 preferred_element_type=jnp.float32) + bcur_ref[...]
    fm = _gn_relu(a1, gcur_ref[...], becur_ref[...], 32)
    a2 = jnp.dot(wlr_ref[...], x,
                 preferred_element_type=jnp.float32) + blr_ref[...]
    feat = _gn_relu(a2, glr_ref[...], belr_ref[...], 32)
    feat_ref[...] = feat
    v = jnp.dot(wprop_ref[:, :pc], col,
                preferred_element_type=jnp.float32) + bprop_ref[...]
    y = (jnp.dot(wprop_ref[:, pc:pc + pc2], feat,
                 preferred_element_type=jnp.float32)
         + jnp.dot(wprop_ref[:, pc + pc2:], fm,
                   preferred_element_type=jnp.float32)
         + v)
    mixed_ref[...] = _gn_relu(y, gprop_ref[...], beprop_ref[...], 32)


def kernel(feature, frame_level_feature, w_cur, b_cur, g_cur, be_cur,
           w_lr, b_lr, g_lr, be_lr, w_prop, b_prop, g_prop, be_prop):
    x = feature[0]
    flf = frame_level_feature[0]
    c, t = x.shape
    pc = w_cur.shape[0]
    pc2 = w_lr.shape[0]
    co = w_prop.shape[0]

    mixed, feat = pl.pallas_call(
        functools.partial(_fused_kernel, t=t, pc=pc, pc2=pc2),
        out_shape=[
            jax.ShapeDtypeStruct((co, t), jnp.float32),
            jax.ShapeDtypeStruct((pc2, t), jnp.float32),
        ],
        compiler_params=pltpu.CompilerParams(vmem_limit_bytes=63 * 2**20),
    )(flf, x, w_cur, w_lr, w_prop,
      b_cur.reshape(-1, 1), g_cur.reshape(-1, 1), be_cur.reshape(-1, 1),
      b_lr.reshape(-1, 1), g_lr.reshape(-1, 1), be_lr.reshape(-1, 1),
      b_prop.reshape(-1, 1), g_prop.reshape(-1, 1), be_prop.reshape(-1, 1))

    return (mixed[None], feat[None])


# speedup vs baseline: 1.0060x; 1.0060x over previous
import functools

import jax
import jax.numpy as jnp
from jax.experimental import pallas as pl
from jax.experimental.pallas import tpu as pltpu

_EPS = 1e-5


def _gn_relu(a, gamma, beta, groups):
    c, tt = a.shape
    gs = c // groups
    rs = jnp.sum(a, axis=1, keepdims=True)
    rq = jnp.sum(a * a, axis=1, keepdims=True)
    gi = jax.lax.broadcasted_iota(jnp.int32, (groups, c), 0)
    gc = jax.lax.broadcasted_iota(jnp.int32, (groups, c), 1) // gs
    gind = (gi == gc).astype(jnp.float32)
    ci = jax.lax.broadcasted_iota(jnp.int32, (c, groups), 0) // gs
    cg = jax.lax.broadcasted_iota(jnp.int32, (c, groups), 1)
    gind_t = (ci == cg).astype(jnp.float32)
    cnt = jnp.float32(gs * tt)
    gmean = jnp.dot(gind, rs, preferred_element_type=jnp.float32) / cnt
    gsq = jnp.dot(gind, rq, preferred_element_type=jnp.float32) / cnt
    ginv = jax.lax.rsqrt(gsq - gmean * gmean + _EPS)
    mean_c = jnp.dot(gind_t, gmean, preferred_element_type=jnp.float32)
    inv_c = jnp.dot(gind_t, ginv, preferred_element_type=jnp.float32)
    sc = gamma * inv_c
    of = beta - mean_c * sc
    return jnp.maximum(a * sc + of, 0.0)


def _fused_kernel(flf_ref, x_ref, wcur_ref, wlr_ref, wprop_ref,
                  bcur_ref, gcur_ref, becur_ref, blr_ref, glr_ref, belr_ref,
                  bprop_ref, gprop_ref, beprop_ref,
                  mixed_ref, feat_ref, *, t, pc, pc2):
    flf = flf_ref[...]
    T = flf.shape[1]
    K = 128
    R = T // K
    m1 = jnp.max(flf, axis=0, keepdims=True)
    m = jnp.concatenate(
        [m1[:, j * K:(j + 1) * K] for j in range(R)], axis=0)
    mn = m / jnp.sum(m)
    ku = jax.lax.broadcasted_iota(jnp.int32, (K, K), 0)
    kv = jax.lax.broadcasted_iota(jnp.int32, (K, K), 1)
    upper = (ku <= kv).astype(jnp.float32)
    rowcum = jnp.dot(mn, upper, preferred_element_type=jnp.float32)
    ru = jax.lax.broadcasted_iota(jnp.int32, (R, R), 0)
    rv = jax.lax.broadcasted_iota(jnp.int32, (R, R), 1)
    strict_lower = (rv < ru).astype(jnp.float32)
    rowtot = jnp.sum(mn, axis=1, keepdims=True)
    prev = jnp.dot(strict_lower, rowtot, preferred_element_type=jnp.float32)
    cdf_i = ((rowcum + prev) * jnp.float32(t)).astype(jnp.int32)
    sentinel = jnp.int32(jnp.iinfo(jnp.int32).max)
    cur = jnp.min(jnp.where(cdf_i >= 0, cdf_i, sentinel))
    lin = (jax.lax.broadcasted_iota(jnp.int32, (R, K), 0) * K
           + jax.lax.broadcasted_iota(jnp.int32, (R, K), 1))
    big = jnp.int32(1 << 30)
    hit = jnp.min(jnp.where(cdf_i == cur, lin, big))
    first_idx = jnp.where(hit == big, jnp.int32(0), hit)
    lin2 = jax.lax.broadcasted_iota(jnp.int32, (T, 1), 0)
    onehot = (lin2 == first_idx).astype(jnp.float32)
    col = jnp.dot(flf, onehot, preferred_element_type=jnp.float32)

    x = x_ref[...].astype(jnp.bfloat16)
    a1 = jnp.dot(wcur_ref[...].astype(jnp.bfloat16), x,
                 preferred_element_type=jnp.float32) + bcur_ref[...]
    fm = _gn_relu(a1, gcur_ref[...], becur_ref[...], 32)
    a2 = jnp.dot(wlr_ref[...].astype(jnp.bfloat16), x,
                 preferred_element_type=jnp.float32) + blr_ref[...]
    feat = _gn_relu(a2, glr_ref[...], belr_ref[...], 32)
    feat_ref[...] = feat
    v = jnp.dot(wprop_ref[:, :pc], col,
                preferred_element_type=jnp.float32) + bprop_ref[...]
    y = (jnp.dot(wprop_ref[:, pc:pc + pc2].astype(jnp.bfloat16),
                 feat.astype(jnp.bfloat16),
                 preferred_element_type=jnp.float32)
         + jnp.dot(wprop_ref[:, pc + pc2:].astype(jnp.bfloat16),
                   fm.astype(jnp.bfloat16),
                   preferred_element_type=jnp.float32)
         + v)
    mixed_ref[...] = _gn_relu(y, gprop_ref[...], beprop_ref[...], 32)


def kernel(feature, frame_level_feature, w_cur, b_cur, g_cur, be_cur,
           w_lr, b_lr, g_lr, be_lr, w_prop, b_prop, g_prop, be_prop):
    x = feature[0]
    flf = frame_level_feature[0]
    c, t = x.shape
    pc = w_cur.shape[0]
    pc2 = w_lr.shape[0]
    co = w_prop.shape[0]

    mixed, feat = pl.pallas_call(
        functools.partial(_fused_kernel, t=t, pc=pc, pc2=pc2),
        out_shape=[
            jax.ShapeDtypeStruct((co, t), jnp.float32),
            jax.ShapeDtypeStruct((pc2, t), jnp.float32),
        ],
        compiler_params=pltpu.CompilerParams(vmem_limit_bytes=63 * 2**20),
    )(flf, x, w_cur, w_lr, w_prop,
      b_cur.reshape(-1, 1), g_cur.reshape(-1, 1), be_cur.reshape(-1, 1),
      b_lr.reshape(-1, 1), g_lr.reshape(-1, 1), be_lr.reshape(-1, 1),
      b_prop.reshape(-1, 1), g_prop.reshape(-1, 1), be_prop.reshape(-1, 1))

    return (mixed[None], feat[None])
